# baseline (speedup 1.0000x reference)
import jax
import jax.numpy as jnp
from jax import lax
from jax.experimental import pallas as pl
from jax.experimental.pallas import tpu as pltpu
from jax.experimental.pallas import tpu_sc as plsc

B = 256
NK = 17
G_TOTAL = B * NK
NW = 32
KP_PER_W = 144
CHUNKS = KP_PER_W // 16
NIDX = KP_PER_W * 9
GCHUNK = 72
NGATHER = NIDX // GCHUNK

SS = (80, 40, 20)
OFFS = (0, 6400, 8000)

_LOG2C = (-2.786812953867443, 5.046876044975941, -3.49249427987935,
          1.5939013634991297, -0.4048671744191854, 0.043428907822139526)
_LN2 = 0.6931471805599453


def _ln_clamped(v):
    bits = lax.bitcast_convert_type(v, jnp.int32)
    e = lax.shift_right_logical(bits, 23) - 127
    m = lax.bitcast_convert_type(
        jnp.bitwise_or(jnp.bitwise_and(bits, 0x7FFFFF), 0x3F800000),
        jnp.float32)
    p = jnp.float32(_LOG2C[5])
    for c in (_LOG2C[4], _LOG2C[3], _LOG2C[2], _LOG2C[1], _LOG2C[0]):
        p = p * m + c
    ln = (e.astype(jnp.float32) + p) * _LN2
    return jnp.maximum(ln, -100.0)


def _sc_body(src, gt2, vis1, out, gtv, visv, idxv, rowsv, outv, sem):
    wid = lax.axis_index("s") * 2 + lax.axis_index("c")
    base = wid * KP_PER_W
    pltpu.sync_copy(gt2, gtv)
    pltpu.sync_copy(vis1, visv)
    lane = lax.iota(jnp.int32, 16)
    for i in range(CHUNKS):
        g = jnp.minimum(base + i * 16 + lane, G_TOTAL - 1)
        x = plsc.load_gather(gtv, [2 * g])
        y = plsc.load_gather(gtv, [2 * g + 1])
        b = lax.shift_right_logical(g * 7711, 17)
        k = g - b * NK
        bterm = (lax.shift_right_logical(b, 7) * 1024
                 + jnp.bitwise_and(b, 127))
        ch0 = 5 + 3 * k
        for si in range(3):
            ss = SS[si]
            inv = float(ss) / 640.0
            ax = jnp.minimum((x * inv).astype(jnp.int32), ss - 1)
            ay = jnp.minimum((y * inv).astype(jnp.int32), ss - 1)
            cell = ax * ss + ay + OFFS[si]
            cterm = (lax.shift_right_logical(cell, 3) * 2048
                     + jnp.bitwise_and(cell, 7) * 128 + bterm)
            for ci in range(3):
                j = si * 3 + ci
                idxv[pl.ds(j * KP_PER_W + i * 16, 16)] = (
                    (ch0 + ci) * 2150400 + cterm)
    copies = []
    for t in range(NGATHER):
        sl = pl.ds(t * GCHUNK, GCHUNK)
        copies.append(pltpu.async_copy(src.at[idxv.at[sl]], rowsv.at[sl], sem))
    for cp in copies:
        cp.wait()

    conf_acc = jnp.zeros((16,), jnp.float32)
    d2_acc = jnp.zeros((16,), jnp.float32)
    nvis_acc = jnp.zeros((16,), jnp.float32)
    for i in range(CHUNKS):
        gr = base + i * 16 + lane
        g = jnp.minimum(gr, G_TOTAL - 1)
        validf = (gr < G_TOTAL).astype(jnp.float32)
        x = plsc.load_gather(gtv, [2 * g])
        y = plsc.load_gather(gtv, [2 * g + 1])
        t = plsc.load_gather(visv, [g])
        maskf = (t > 0.0).astype(jnp.float32) * validf
        nvis_acc = nvis_acc + maskf
        for si in range(3):
            px = rowsv[pl.ds((si * 3 + 0) * KP_PER_W + i * 16, 16)]
            py = rowsv[pl.ds((si * 3 + 1) * KP_PER_W + i * 16, 16)]
            pc = rowsv[pl.ds((si * 3 + 2) * KP_PER_W + i * 16, 16)]
            conf_acc = conf_acc - validf * (
                t * _ln_clamped(pc) + (1.0 - t) * _ln_clamped(1.0 - pc))
            dx = px - x
            dy = py - y
            d2_acc = d2_acc + maskf * (dx * dx + dy * dy)
    outv[pl.ds(0, 16)] = conf_acc
    outv[pl.ds(16, 16)] = d2_acc
    outv[pl.ds(32, 16)] = nvis_acc
    pltpu.sync_copy(outv, out.at[wid])


_sc_loss = pl.kernel(
    _sc_body,
    out_type=jax.ShapeDtypeStruct((NW, 48), jnp.float32),
    mesh=plsc.VectorSubcoreMesh(core_axis_name="c", subcore_axis_name="s"),
    scratch_types=[
        pltpu.VMEM((2 * G_TOTAL,), jnp.float32),
        pltpu.VMEM((G_TOTAL,), jnp.float32),
        pltpu.VMEM((NIDX,), jnp.int32),
        pltpu.VMEM((NIDX,), jnp.float32),
        pltpu.VMEM((48,), jnp.float32),
        pltpu.SemaphoreType.DMA,
    ],
    compiler_params=pltpu.CompilerParams(needs_layout_passes=False),
)


def _tc_final_body(m_ref, o_ref):
    m = m_ref[...]
    conf = jnp.sum(m[:, 0:16])
    d2 = jnp.sum(m[:, 16:32])
    nvis = jnp.sum(m[:, 32:48])
    o_ref[0, 0] = conf / G_TOTAL + d2 / (2.0 * nvis + 1e-6)


_tc_final = pl.pallas_call(
    _tc_final_body,
    out_shape=jax.ShapeDtypeStruct((1, 1), jnp.float32),
    in_specs=[pl.BlockSpec(memory_space=pltpu.VMEM)],
    out_specs=pl.BlockSpec(memory_space=pltpu.SMEM),
)


@jax.jit
def kernel(output, gt_keypoints, keypoint_visibility):
    src = (output.transpose(1, 2, 0)
           .reshape(58800, 8, 2, 128)
           .transpose(0, 2, 1, 3)
           .reshape(-1))
    partials = _sc_loss(src,
                        gt_keypoints.reshape(-1),
                        keypoint_visibility.reshape(-1))
    return _tc_final(partials)[0, 0]


# speedup vs baseline: 1.0543x; 1.0543x over previous
import jax
import jax.numpy as jnp
from jax import lax
from jax.experimental import pallas as pl
from jax.experimental.pallas import tpu as pltpu
from jax.experimental.pallas import tpu_sc as plsc

B = 256
NK = 17
G_TOTAL = B * NK
NW = 32
KP_PER_W = 144
G_PAD = NW * KP_PER_W
CHUNKS = KP_PER_W // 16
NIDX = KP_PER_W * 9

SS = (80, 40, 20)
OFFS = (0, 6400, 8000)

GATHER_SLICES = [(i * 128, 128) for i in range(10)] + [(1280, 16)]

_LOG2C = (-2.786812953867443, 5.046876044975941, -3.49249427987935,
          1.5939013634991297, -0.4048671744191854, 0.043428907822139526)
_LN2 = 0.6931471805599453


def _ln_clamped(v):
    bits = lax.bitcast_convert_type(v, jnp.int32)
    e = lax.shift_right_logical(bits, 23) - 127
    m = lax.bitcast_convert_type(
        jnp.bitwise_or(jnp.bitwise_and(bits, 0x7FFFFF), 0x3F800000),
        jnp.float32)
    p = jnp.float32(_LOG2C[5])
    for c in (_LOG2C[4], _LOG2C[3], _LOG2C[2], _LOG2C[1], _LOG2C[0]):
        p = p * m + c
    ln = (e.astype(jnp.float32) + p) * _LN2
    return jnp.maximum(ln, -100.0)


def _sc_body(src, gt2, vis1, out, gxy, vv, idxv, rowsv, outv, sem):
    wid = lax.axis_index("s") * 2 + lax.axis_index("c")
    base = wid * KP_PER_W
    pltpu.sync_copy(gt2.at[pl.ds(2 * base, 2 * KP_PER_W)], gxy)
    pltpu.sync_copy(vis1.at[pl.ds(base, KP_PER_W)], vv)
    lane = lax.iota(jnp.int32, 16)
    for i in range(CHUNKS):
        ll = i * 16 + lane
        g = jnp.minimum(base + ll, G_TOTAL - 1)
        x = plsc.load_gather(gxy, [2 * ll])
        y = plsc.load_gather(gxy, [2 * ll + 1])
        b = lax.shift_right_logical(g * 7711, 17)
        k = g - b * NK
        bterm = (lax.shift_right_logical(b, 7) * 1024
                 + jnp.bitwise_and(b, 127))
        ch0 = 5 + 3 * k
        for si in range(3):
            ss = SS[si]
            inv = float(ss) / 640.0
            ax = jnp.minimum((x * inv).astype(jnp.int32), ss - 1)
            ay = jnp.minimum((y * inv).astype(jnp.int32), ss - 1)
            cell = ax * ss + ay + OFFS[si]
            cterm = (lax.shift_right_logical(cell, 3) * 2048
                     + jnp.bitwise_and(cell, 7) * 128 + bterm)
            for ci in range(3):
                j = si * 3 + ci
                idxv[pl.ds(j * KP_PER_W + i * 16, 16)] = (
                    (ch0 + ci) * 2150400 + cterm)
    copies = []
    for off, n in GATHER_SLICES:
        sl = pl.ds(off, n)
        copies.append(pltpu.async_copy(src.at[idxv.at[sl]], rowsv.at[sl], sem))
    for cp in copies:
        cp.wait()

    conf_acc = jnp.zeros((16,), jnp.float32)
    d2_acc = jnp.zeros((16,), jnp.float32)
    nvis_acc = jnp.zeros((16,), jnp.float32)
    for i in range(CHUNKS):
        ll = i * 16 + lane
        validf = (base + ll < G_TOTAL).astype(jnp.float32)
        x = plsc.load_gather(gxy, [2 * ll])
        y = plsc.load_gather(gxy, [2 * ll + 1])
        t = plsc.load_gather(vv, [ll])
        maskf = (t > 0.0).astype(jnp.float32) * validf
        nvis_acc = nvis_acc + maskf
        for si in range(3):
            px = rowsv[pl.ds((si * 3 + 0) * KP_PER_W + i * 16, 16)]
            py = rowsv[pl.ds((si * 3 + 1) * KP_PER_W + i * 16, 16)]
            pc = rowsv[pl.ds((si * 3 + 2) * KP_PER_W + i * 16, 16)]
            conf_acc = conf_acc - validf * (
                t * _ln_clamped(pc) + (1.0 - t) * _ln_clamped(1.0 - pc))
            dx = px - x
            dy = py - y
            d2_acc = d2_acc + maskf * (dx * dx + dy * dy)
    outv[pl.ds(0, 16)] = conf_acc
    outv[pl.ds(16, 16)] = d2_acc
    outv[pl.ds(32, 16)] = nvis_acc
    pltpu.sync_copy(outv, out.at[wid])


_sc_loss = pl.kernel(
    _sc_body,
    out_type=jax.ShapeDtypeStruct((NW, 48), jnp.float32),
    mesh=plsc.VectorSubcoreMesh(core_axis_name="c", subcore_axis_name="s"),
    scratch_types=[
        pltpu.VMEM((2 * KP_PER_W,), jnp.float32),
        pltpu.VMEM((KP_PER_W,), jnp.float32),
        pltpu.VMEM((NIDX,), jnp.int32),
        pltpu.VMEM((NIDX,), jnp.float32),
        pltpu.VMEM((48,), jnp.float32),
        pltpu.SemaphoreType.DMA,
    ],
    compiler_params=pltpu.CompilerParams(needs_layout_passes=False),
)


def _tc_final_body(m_ref, o_ref):
    m = m_ref[...]
    conf = jnp.sum(m[:, 0:16])
    d2 = jnp.sum(m[:, 16:32])
    nvis = jnp.sum(m[:, 32:48])
    o_ref[0, 0] = conf / G_TOTAL + d2 / (2.0 * nvis + 1e-6)


_tc_final = pl.pallas_call(
    _tc_final_body,
    out_shape=jax.ShapeDtypeStruct((1, 1), jnp.float32),
    in_specs=[pl.BlockSpec(memory_space=pltpu.VMEM)],
    out_specs=pl.BlockSpec(memory_space=pltpu.SMEM),
)


@jax.jit
def kernel(output, gt_keypoints, keypoint_visibility):
    src = (output.transpose(1, 2, 0)
           .reshape(58800, 8, 2, 128)
           .transpose(0, 2, 1, 3)
           .reshape(-1))
    gt2 = jnp.pad(gt_keypoints.reshape(-1), (0, 2 * (G_PAD - G_TOTAL)))
    vis1 = jnp.pad(keypoint_visibility.reshape(-1), (0, G_PAD - G_TOTAL))
    partials = _sc_loss(src, gt2, vis1)
    return _tc_final(partials)[0, 0]


# speedup vs baseline: 1.1088x; 1.0517x over previous
import jax
import jax.numpy as jnp
from jax import lax
from jax.experimental import pallas as pl
from jax.experimental.pallas import tpu as pltpu
from jax.experimental.pallas import tpu_sc as plsc

B = 256
NK = 17
G_TOTAL = B * NK
NW = 32
KP_PER_W = 144
G_PAD = NW * KP_PER_W
CHUNKS = KP_PER_W // 16
NIDX = KP_PER_W * 9

SS = (80, 40, 20)
OFFS = (0, 6400, 8000)


_LOG2C = (-2.786812953867443, 5.046876044975941, -3.49249427987935,
          1.5939013634991297, -0.4048671744191854, 0.043428907822139526)
_LN2 = 0.6931471805599453


def _ln_clamped(v):
    bits = lax.bitcast_convert_type(v, jnp.int32)
    e = lax.shift_right_logical(bits, 23) - 127
    m = lax.bitcast_convert_type(
        jnp.bitwise_or(jnp.bitwise_and(bits, 0x7FFFFF), 0x3F800000),
        jnp.float32)
    p = jnp.float32(_LOG2C[5])
    for c in (_LOG2C[4], _LOG2C[3], _LOG2C[2], _LOG2C[1], _LOG2C[0]):
        p = p * m + c
    ln = (e.astype(jnp.float32) + p) * _LN2
    return jnp.maximum(ln, -100.0)


def _sc_body(src, gt2, vis1, out, gxy, vv, idxv, rowsv, outv, sem):
    wid = lax.axis_index("s") * 2 + lax.axis_index("c")
    base = wid * KP_PER_W
    pltpu.sync_copy(gt2.at[pl.ds(2 * base, 2 * KP_PER_W)], gxy)
    pltpu.sync_copy(vis1.at[pl.ds(base, KP_PER_W)], vv)
    lane = lax.iota(jnp.int32, 16)
    copies = []
    for i in range(CHUNKS):
        ll = i * 16 + lane
        g = jnp.minimum(base + ll, G_TOTAL - 1)
        x = plsc.load_gather(gxy, [2 * ll])
        y = plsc.load_gather(gxy, [2 * ll + 1])
        b = lax.shift_right_logical(g * 7711, 17)
        k = g - b * NK
        bterm = (lax.shift_right_logical(b, 7) * 1024
                 + jnp.bitwise_and(b, 127))
        ch0 = 5 + 3 * k
        for si in range(3):
            ss = SS[si]
            inv = float(ss) / 640.0
            ax = jnp.minimum((x * inv).astype(jnp.int32), ss - 1)
            ay = jnp.minimum((y * inv).astype(jnp.int32), ss - 1)
            cell = ax * ss + ay + OFFS[si]
            cterm = (lax.shift_right_logical(cell, 3) * 2048
                     + jnp.bitwise_and(cell, 7) * 128 + bterm)
            for ci in range(3):
                j = si * 3 + ci
                idxv[pl.ds(i * 144 + j * 16, 16)] = (
                    (ch0 + ci) * 2150400 + cterm)
        for half in range(2):
            sl = pl.ds(i * 144 + half * 72, 72)
            copies.append(
                pltpu.async_copy(src.at[idxv.at[sl]], rowsv.at[sl], sem))
    for cp in copies:
        cp.wait()

    conf_acc = jnp.zeros((16,), jnp.float32)
    d2_acc = jnp.zeros((16,), jnp.float32)
    nvis_acc = jnp.zeros((16,), jnp.float32)
    for i in range(CHUNKS):
        ll = i * 16 + lane
        validf = (base + ll < G_TOTAL).astype(jnp.float32)
        x = plsc.load_gather(gxy, [2 * ll])
        y = plsc.load_gather(gxy, [2 * ll + 1])
        t = plsc.load_gather(vv, [ll])
        maskf = (t > 0.0).astype(jnp.float32) * validf
        nvis_acc = nvis_acc + maskf
        for si in range(3):
            px = rowsv[pl.ds(i * 144 + (si * 3 + 0) * 16, 16)]
            py = rowsv[pl.ds(i * 144 + (si * 3 + 1) * 16, 16)]
            pc = rowsv[pl.ds(i * 144 + (si * 3 + 2) * 16, 16)]
            lnp = _ln_clamped(pc)
            ln1mp = _ln_clamped(1.0 - pc)
            conf_acc = conf_acc - validf * (ln1mp + t * (lnp - ln1mp))
            dx = px - x
            dy = py - y
            d2_acc = d2_acc + maskf * (dx * dx + dy * dy)
    outv[pl.ds(0, 16)] = conf_acc
    outv[pl.ds(16, 16)] = d2_acc
    outv[pl.ds(32, 16)] = nvis_acc
    pltpu.sync_copy(outv, out.at[wid])


_sc_loss = pl.kernel(
    _sc_body,
    out_type=jax.ShapeDtypeStruct((NW, 48), jnp.float32),
    mesh=plsc.VectorSubcoreMesh(core_axis_name="c", subcore_axis_name="s"),
    scratch_types=[
        pltpu.VMEM((2 * KP_PER_W,), jnp.float32),
        pltpu.VMEM((KP_PER_W,), jnp.float32),
        pltpu.VMEM((NIDX,), jnp.int32),
        pltpu.VMEM((NIDX,), jnp.float32),
        pltpu.VMEM((48,), jnp.float32),
        pltpu.SemaphoreType.DMA,
    ],
    compiler_params=pltpu.CompilerParams(needs_layout_passes=False),
)


def _tc_final_body(m_ref, o_ref):
    m = m_ref[...]
    conf = jnp.sum(m[:, 0:16])
    d2 = jnp.sum(m[:, 16:32])
    nvis = jnp.sum(m[:, 32:48])
    o_ref[0, 0] = conf / G_TOTAL + d2 / (2.0 * nvis + 1e-6)


_tc_final = pl.pallas_call(
    _tc_final_body,
    out_shape=jax.ShapeDtypeStruct((1, 1), jnp.float32),
    in_specs=[pl.BlockSpec(memory_space=pltpu.VMEM)],
    out_specs=pl.BlockSpec(memory_space=pltpu.SMEM),
)


@jax.jit
def kernel(output, gt_keypoints, keypoint_visibility):
    src = (output.transpose(1, 2, 0)
           .reshape(58800, 8, 2, 128)
           .transpose(0, 2, 1, 3)
           .reshape(-1))
    gt2 = jnp.pad(gt_keypoints.reshape(-1), (0, 2 * (G_PAD - G_TOTAL)))
    vis1 = jnp.pad(keypoint_visibility.reshape(-1), (0, G_PAD - G_TOTAL))
    partials = _sc_loss(src, gt2, vis1)
    return _tc_final(partials)[0, 0]


# speedup vs baseline: 1.1276x; 1.0170x over previous
import jax
import jax.numpy as jnp
from jax import lax
from jax.experimental import pallas as pl
from jax.experimental.pallas import tpu as pltpu
from jax.experimental.pallas import tpu_sc as plsc

B = 256
NK = 17
G_TOTAL = B * NK
NW = 32
KP_PER_W = 144
G_PAD = NW * KP_PER_W
CHUNKS = KP_PER_W // 16
NIDX = KP_PER_W * 9

SS = (80, 40, 20)
OFFS = (0, 6400, 8000)


_LOG2C = (-2.786812953867443, 5.046876044975941, -3.49249427987935,
          1.5939013634991297, -0.4048671744191854, 0.043428907822139526)
_LN2 = 0.6931471805599453


def _ln_clamped(v):
    bits = lax.bitcast_convert_type(v, jnp.int32)
    e = lax.shift_right_logical(bits, 23) - 127
    m = lax.bitcast_convert_type(
        jnp.bitwise_or(jnp.bitwise_and(bits, 0x7FFFFF), 0x3F800000),
        jnp.float32)
    p = jnp.float32(_LOG2C[5])
    for c in (_LOG2C[4], _LOG2C[3], _LOG2C[2], _LOG2C[1], _LOG2C[0]):
        p = p * m + c
    ln = (e.astype(jnp.float32) + p) * _LN2
    return jnp.maximum(ln, -100.0)


def _sc_body(src, gt2, vis1, out, gxy, vv, idxv, rowsv, outv, sem):
    wid = lax.axis_index("s") * 2 + lax.axis_index("c")
    base = wid * KP_PER_W
    pltpu.sync_copy(gt2.at[pl.ds(2 * base, 2 * KP_PER_W)], gxy)
    pltpu.sync_copy(vis1.at[pl.ds(base, KP_PER_W)], vv)
    lane = lax.iota(jnp.int32, 16)
    copies = []
    for i in range(CHUNKS):
        ll = i * 16 + lane
        g = jnp.minimum(base + ll, G_TOTAL - 1)
        x = plsc.load_gather(gxy, [2 * ll])
        y = plsc.load_gather(gxy, [2 * ll + 1])
        b = lax.shift_right_logical(g * 7711, 17)
        k = g - b * NK
        bterm = (lax.shift_right_logical(b, 7) * 1024
                 + jnp.bitwise_and(b, 127))
        ch0 = 5 + 3 * k
        for si in range(3):
            ss = SS[si]
            inv = float(ss) / 640.0
            ax = jnp.minimum((x * inv).astype(jnp.int32), ss - 1)
            ay = jnp.minimum((y * inv).astype(jnp.int32), ss - 1)
            cell = ax * ss + ay + OFFS[si]
            cterm = (lax.shift_right_logical(cell, 3) * 2048
                     + jnp.bitwise_and(cell, 7) * 128 + bterm)
            for ci in range(3):
                j = si * 3 + ci
                idxv[pl.ds(i * 144 + j * 16, 16)] = (
                    (ch0 + ci) * 2150400 + cterm)
        for half in range(2):
            sl = pl.ds(i * 144 + half * 72, 72)
            copies.append(
                pltpu.async_copy(src.at[idxv.at[sl]], rowsv.at[sl], sem))
    for cp in copies:
        cp.wait()

    def _chunk(i, accs):
        conf_acc, d2_acc, nvis_acc = accs
        ll = i * 16 + lane
        validf = (base + ll < G_TOTAL).astype(jnp.float32)
        x = plsc.load_gather(gxy, [2 * ll])
        y = plsc.load_gather(gxy, [2 * ll + 1])
        t = plsc.load_gather(vv, [ll])
        maskf = (t > 0.0).astype(jnp.float32) * validf
        nvis_acc = nvis_acc + maskf
        for si in range(3):
            px = rowsv[pl.ds(i * 144 + (si * 3 + 0) * 16, 16)]
            py = rowsv[pl.ds(i * 144 + (si * 3 + 1) * 16, 16)]
            pc = rowsv[pl.ds(i * 144 + (si * 3 + 2) * 16, 16)]
            lnp = _ln_clamped(pc)
            ln1mp = _ln_clamped(1.0 - pc)
            conf_acc = conf_acc - validf * (ln1mp + t * (lnp - ln1mp))
            dx = px - x
            dy = py - y
            d2_acc = d2_acc + maskf * (dx * dx + dy * dy)
        return conf_acc, d2_acc, nvis_acc

    zero = jnp.zeros((16,), jnp.float32)
    conf_acc, d2_acc, nvis_acc = lax.fori_loop(
        0, CHUNKS, _chunk, (zero, zero, zero))
    outv[pl.ds(0, 16)] = conf_acc
    outv[pl.ds(16, 16)] = d2_acc
    outv[pl.ds(32, 16)] = nvis_acc
    pltpu.sync_copy(outv, out.at[wid])


_sc_loss = pl.kernel(
    _sc_body,
    out_type=jax.ShapeDtypeStruct((NW, 48), jnp.float32),
    mesh=plsc.VectorSubcoreMesh(core_axis_name="c", subcore_axis_name="s"),
    scratch_types=[
        pltpu.VMEM((2 * KP_PER_W,), jnp.float32),
        pltpu.VMEM((KP_PER_W,), jnp.float32),
        pltpu.VMEM((NIDX,), jnp.int32),
        pltpu.VMEM((NIDX,), jnp.float32),
        pltpu.VMEM((48,), jnp.float32),
        pltpu.SemaphoreType.DMA,
    ],
    compiler_params=pltpu.CompilerParams(needs_layout_passes=False),
)


def _tc_final_body(m_ref, o_ref):
    m = m_ref[...]
    conf = jnp.sum(m[:, 0:16])
    d2 = jnp.sum(m[:, 16:32])
    nvis = jnp.sum(m[:, 32:48])
    o_ref[0, 0] = conf / G_TOTAL + d2 / (2.0 * nvis + 1e-6)


_tc_final = pl.pallas_call(
    _tc_final_body,
    out_shape=jax.ShapeDtypeStruct((1, 1), jnp.float32),
    in_specs=[pl.BlockSpec(memory_space=pltpu.VMEM)],
    out_specs=pl.BlockSpec(memory_space=pltpu.SMEM),
)


@jax.jit
def kernel(output, gt_keypoints, keypoint_visibility):
    src = (output.transpose(1, 2, 0)
           .reshape(58800, 8, 2, 128)
           .transpose(0, 2, 1, 3)
           .reshape(-1))
    gt2 = jnp.pad(gt_keypoints.reshape(-1), (0, 2 * (G_PAD - G_TOTAL)))
    vis1 = jnp.pad(keypoint_visibility.reshape(-1), (0, G_PAD - G_TOTAL))
    partials = _sc_loss(src, gt2, vis1)
    return _tc_final(partials)[0, 0]


# speedup vs baseline: 1.1392x; 1.0103x over previous
import jax
import jax.numpy as jnp
from jax import lax
from jax.experimental import pallas as pl
from jax.experimental.pallas import tpu as pltpu
from jax.experimental.pallas import tpu_sc as plsc

B = 256
NK = 17
G_TOTAL = B * NK
NW = 32
KP_PER_W = 144
G_PAD = NW * KP_PER_W
CHUNKS = KP_PER_W // 16
NIDX = KP_PER_W * 9

SS = (80, 40, 20)
OFFS = (0, 6400, 8000)


_LOG2C = (-2.786812953867443, 5.046876044975941, -3.49249427987935,
          1.5939013634991297, -0.4048671744191854, 0.043428907822139526)
_LN2 = 0.6931471805599453


def _ln_clamped(v):
    bits = lax.bitcast_convert_type(v, jnp.int32)
    e = lax.shift_right_logical(bits, 23) - 127
    m = lax.bitcast_convert_type(
        jnp.bitwise_or(jnp.bitwise_and(bits, 0x7FFFFF), 0x3F800000),
        jnp.float32)
    p = jnp.float32(_LOG2C[5])
    for c in (_LOG2C[4], _LOG2C[3], _LOG2C[2], _LOG2C[1], _LOG2C[0]):
        p = p * m + c
    ln = (e.astype(jnp.float32) + p) * _LN2
    return jnp.maximum(ln, -100.0)


def _sc_body(src, gt2, vis1, out, gxy, vv, idxv, rowsv, outv, sem):
    wid = lax.axis_index("s") * 2 + lax.axis_index("c")
    base = wid * KP_PER_W
    pltpu.sync_copy(gt2.at[pl.ds(2 * base, 2 * KP_PER_W)], gxy)
    pltpu.sync_copy(vis1.at[pl.ds(base, KP_PER_W)], vv)
    lane = lax.iota(jnp.int32, 16)

    def _idx_chunk(i, carry):
        ll = i * 16 + lane
        g = jnp.minimum(base + ll, G_TOTAL - 1)
        x = plsc.load_gather(gxy, [2 * ll])
        y = plsc.load_gather(gxy, [2 * ll + 1])
        b = lax.shift_right_logical(g * 7711, 17)
        k = g - b * NK
        bterm = (lax.shift_right_logical(b, 7) * 1024
                 + jnp.bitwise_and(b, 127))
        ch0 = 5 + 3 * k
        for si in range(3):
            ss = SS[si]
            inv = float(ss) / 640.0
            ax = jnp.minimum((x * inv).astype(jnp.int32), ss - 1)
            ay = jnp.minimum((y * inv).astype(jnp.int32), ss - 1)
            cell = ax * ss + ay + OFFS[si]
            cterm = (lax.shift_right_logical(cell, 3) * 2048
                     + jnp.bitwise_and(cell, 7) * 128 + bterm)
            for ci in range(3):
                j = si * 3 + ci
                idxv[pl.ds(i * 144 + j * 16, 16)] = (
                    (ch0 + ci) * 2150400 + cterm)
        for half in range(2):
            sl = pl.ds(i * 144 + half * 72, 72)
            pltpu.async_copy(src.at[idxv.at[sl]], rowsv.at[sl], sem)
        return carry

    lax.fori_loop(0, CHUNKS, _idx_chunk, 0)
    pltpu.make_async_copy(src.at[pl.ds(0, NIDX)], rowsv, sem).wait()

    def _chunk(i, accs):
        conf_acc, d2_acc, nvis_acc = accs
        ll = i * 16 + lane
        validf = (base + ll < G_TOTAL).astype(jnp.float32)
        x = plsc.load_gather(gxy, [2 * ll])
        y = plsc.load_gather(gxy, [2 * ll + 1])
        t = plsc.load_gather(vv, [ll])
        maskf = (t > 0.0).astype(jnp.float32) * validf
        nvis_acc = nvis_acc + maskf
        for si in range(3):
            px = rowsv[pl.ds(i * 144 + (si * 3 + 0) * 16, 16)]
            py = rowsv[pl.ds(i * 144 + (si * 3 + 1) * 16, 16)]
            pc = rowsv[pl.ds(i * 144 + (si * 3 + 2) * 16, 16)]
            lnp = _ln_clamped(pc)
            ln1mp = _ln_clamped(1.0 - pc)
            conf_acc = conf_acc - validf * (ln1mp + t * (lnp - ln1mp))
            dx = px - x
            dy = py - y
            d2_acc = d2_acc + maskf * (dx * dx + dy * dy)
        return conf_acc, d2_acc, nvis_acc

    zero = jnp.zeros((16,), jnp.float32)
    conf_acc, d2_acc, nvis_acc = lax.fori_loop(
        0, CHUNKS, _chunk, (zero, zero, zero))
    outv[pl.ds(0, 16)] = conf_acc
    outv[pl.ds(16, 16)] = d2_acc
    outv[pl.ds(32, 16)] = nvis_acc
    pltpu.sync_copy(outv, out.at[wid])


_sc_loss = pl.kernel(
    _sc_body,
    out_type=jax.ShapeDtypeStruct((NW, 48), jnp.float32),
    mesh=plsc.VectorSubcoreMesh(core_axis_name="c", subcore_axis_name="s"),
    scratch_types=[
        pltpu.VMEM((2 * KP_PER_W,), jnp.float32),
        pltpu.VMEM((KP_PER_W,), jnp.float32),
        pltpu.VMEM((NIDX,), jnp.int32),
        pltpu.VMEM((NIDX,), jnp.float32),
        pltpu.VMEM((48,), jnp.float32),
        pltpu.SemaphoreType.DMA,
    ],
    compiler_params=pltpu.CompilerParams(needs_layout_passes=False),
)


def _tc_final_body(m_ref, o_ref):
    m = m_ref[...]
    conf = jnp.sum(m[:, 0:16])
    d2 = jnp.sum(m[:, 16:32])
    nvis = jnp.sum(m[:, 32:48])
    o_ref[0, 0] = conf / G_TOTAL + d2 / (2.0 * nvis + 1e-6)


_tc_final = pl.pallas_call(
    _tc_final_body,
    out_shape=jax.ShapeDtypeStruct((1, 1), jnp.float32),
    in_specs=[pl.BlockSpec(memory_space=pltpu.VMEM)],
    out_specs=pl.BlockSpec(memory_space=pltpu.SMEM),
)


@jax.jit
def kernel(output, gt_keypoints, keypoint_visibility):
    src = (output.transpose(1, 2, 0)
           .reshape(58800, 8, 2, 128)
           .transpose(0, 2, 1, 3)
           .reshape(-1))
    gt2 = jnp.pad(gt_keypoints.reshape(-1), (0, 2 * (G_PAD - G_TOTAL)))
    vis1 = jnp.pad(keypoint_visibility.reshape(-1), (0, G_PAD - G_TOTAL))
    partials = _sc_loss(src, gt2, vis1)
    return _tc_final(partials)[0, 0]
